# Initial kernel scaffold; baseline (speedup 1.0000x reference)
import jax
import jax.numpy as jnp
from jax.experimental import pallas as pl


def kernel(x, edge_index, edge_weight, W1, b1, W2, W3, b3):
    raise NotImplementedError("write your pallas kernel here")


# speedup vs baseline: 6.4649x; 6.4649x over previous
import functools

import jax
import jax.numpy as jnp
from jax import lax
from jax.experimental import pallas as pl
from jax.experimental.pallas import tpu as pltpu
from jax.experimental.pallas import tpu_sc as plsc

N = 10000
NP = 10240
NQ = NP // 8
D = 128
E = 320000
EROWS = E // 128
NC = 2
NS = 16
NW = NC * NS
RPT = NP // NS
QPT = NQ // NS
NGROUPS = EROWS // 8
TAIL_ROWS = EROWS - NGROUPS * 8
BASE_GROUPS = NGROUPS // NW
EXTRA_GROUPS = NGROUPS - BASE_GROUPS * NW

BN = 1024
QBN = BN // 8


def _dense_body(x_ref, w1_ref, b1_ref, w2_ref, w3_ref, b3_ref,
                a_ref, xw2_ref, xw3c_ref):
    xb = x_ref[...]
    a_ref[...] = (jnp.dot(xb, w1_ref[...], preferred_element_type=jnp.float32)
                  + b1_ref[...])
    xw2_ref[...] = jnp.dot(xb, w2_ref[...], preferred_element_type=jnp.float32)
    xw3c_ref[...] = (jnp.dot(xb, w3_ref[...], preferred_element_type=jnp.float32)
                     + b3_ref[...])


def _dense(xp, W1, b1, W2, W3, b3):
    grid = (NP // BN,)
    return pl.pallas_call(
        _dense_body,
        grid=grid,
        in_specs=[
            pl.BlockSpec((BN, D), lambda i: (i, 0)),
            pl.BlockSpec((D, D), lambda i: (0, 0)),
            pl.BlockSpec((1, D), lambda i: (0, 0)),
            pl.BlockSpec((D, D), lambda i: (0, 0)),
            pl.BlockSpec((D, D), lambda i: (0, 0)),
            pl.BlockSpec((1, D), lambda i: (0, 0)),
        ],
        out_specs=[
            pl.BlockSpec((BN, D), lambda i: (i, 0)),
            pl.BlockSpec((BN, D), lambda i: (i, 0)),
            pl.BlockSpec((BN, D), lambda i: (i, 0)),
        ],
        out_shape=[
            jax.ShapeDtypeStruct((NP, D), jnp.float32),
            jax.ShapeDtypeStruct((NP, D), jnp.float32),
            jax.ShapeDtypeStruct((NP, D), jnp.float32),
        ],
    )(xp, W1, b1.reshape(1, D), W2, W3, b3.reshape(1, D))


def _sc_body(a_hbm, src_hbm, dst_hbm, w_hbm, acc_out, wacc_out,
             srcix_v, dstix_v, dstq_v, w_v, rows_v, wrow_v, acc_s, wacc_s,
             sem):
    c = lax.axis_index("c")
    s = lax.axis_index("s")
    wid = c * NS + s

    def zbody(e, carry):
        for cg in range(8):
            sl = pl.ds(cg * 16, 16)
            rows_v[e, sl] = jnp.zeros((16,), jnp.float32)
            wrow_v[e, sl] = jnp.zeros((16,), jnp.float32)
        return carry

    lax.fori_loop(0, 128, zbody, 0)

    zbase = s * RPT
    for k in range(RPT // 128):
        pltpu.sync_copy(rows_v, acc_s.at[pl.ds(zbase + 128 * k, 128)])
    qbase = s * QPT
    pltpu.sync_copy(rows_v.at[pl.ds(0, QPT)], wacc_s.at[pl.ds(qbase, QPT)])
    plsc.subcore_barrier()

    gstart = wid * BASE_GROUPS + jnp.minimum(wid, EXTRA_GROUPS)
    ngroups = BASE_GROUPS + jnp.where(wid < EXTRA_GROUPS, 1, 0)

    def do_chunk(k):
        pltpu.async_copy(a_hbm.at[srcix_v.at[k]], rows_v, sem).wait()

        def ebody(g, ecarry):
            wv = w_v[k, pl.ds(g * 16, 16)]
            dv = dstix_v[k, pl.ds(g * 16, 16)]
            for r in range(16):
                ws = wv[r]
                e = g * 16 + r
                for cg in range(8):
                    sl = pl.ds(cg * 16, 16)
                    rows_v[e, sl] = rows_v[e, sl] * ws
                off = (dv[r] & 7) * 16
                wrow_v[e, pl.ds(off, 16)] = jnp.full((16,), ws, jnp.float32)
            return ecarry

        lax.fori_loop(0, 8, ebody, 0)
        pltpu.sync_copy(rows_v, acc_s.at[dstix_v.at[k]], add=True)
        pltpu.sync_copy(wrow_v, wacc_s.at[dstq_v.at[k]], add=True)

        def rezero(g, ecarry):
            dv = dstix_v[k, pl.ds(g * 16, 16)]
            for r in range(16):
                off = (dv[r] & 7) * 16
                wrow_v[g * 16 + r, pl.ds(off, 16)] = jnp.zeros((16,), jnp.float32)
            return ecarry

        lax.fori_loop(0, 8, rezero, 0)

    def load_group(grow, nk):
        pltpu.sync_copy(src_hbm.at[pl.ds(grow, nk)],
                        srcix_v.at[pl.ds(0, nk)])
        pltpu.sync_copy(dst_hbm.at[pl.ds(grow, nk)],
                        dstix_v.at[pl.ds(0, nk)])
        pltpu.sync_copy(w_hbm.at[pl.ds(grow, nk)], w_v.at[pl.ds(0, nk)])
        for k in range(nk):
            for g in range(8):
                sl = pl.ds(g * 16, 16)
                dstq_v[k, sl] = lax.shift_right_logical(dstix_v[k, sl], 3)

    def group_body(j, carry):
        grow = pl.multiple_of((gstart + j) * 8, 8)
        load_group(grow, 8)
        for k in range(8):
            do_chunk(k)
        return carry

    lax.fori_loop(0, ngroups, group_body, 0)

    if TAIL_ROWS:
        @pl.when(wid == NW - 1)
        def _tail():
            load_group(NGROUPS * 8, TAIL_ROWS)
            for k in range(TAIL_ROWS):
                do_chunk(k)

    plsc.subcore_barrier()

    pltpu.sync_copy(acc_s.at[pl.ds(zbase, RPT)],
                    acc_out.at[c, pl.ds(zbase, RPT)])
    pltpu.sync_copy(wacc_s.at[pl.ds(qbase, QPT)],
                    wacc_out.at[c, pl.ds(qbase, QPT)])


def _make_sc_aggregate():
    mesh = plsc.VectorSubcoreMesh(core_axis_name="c", subcore_axis_name="s")
    return functools.partial(
        pl.kernel,
        mesh=mesh,
        out_type=[
            jax.ShapeDtypeStruct((NC, NP, D), jnp.float32),
            jax.ShapeDtypeStruct((NC, NQ, D), jnp.float32),
        ],
        scratch_types=[
            pltpu.VMEM((8, 128), jnp.int32),
            pltpu.VMEM((8, 128), jnp.int32),
            pltpu.VMEM((8, 128), jnp.int32),
            pltpu.VMEM((8, 128), jnp.float32),
            pltpu.VMEM((128, D), jnp.float32),
            pltpu.VMEM((128, D), jnp.float32),
            pltpu.VMEM_SHARED((NP, D), jnp.float32),
            pltpu.VMEM_SHARED((NQ, D), jnp.float32),
            pltpu.SemaphoreType.DMA,
        ],
    )


def _combine_body(acc_ref, wacc_ref, xw2_ref, xw3c_ref, o_ref):
    acc = acc_ref[0] + acc_ref[1]
    w8 = wacc_ref[0] + wacc_ref[1]
    r_ix = lax.broadcasted_iota(jnp.int32, (BN, D), 0)
    l_ix = lax.broadcasted_iota(jnp.int32, (BN, D), 1)
    q_sel = (l_ix == r_ix // 8).astype(jnp.float32)
    row_exp = jnp.dot(q_sel[:, :QBN], w8,
                      preferred_element_type=jnp.float32)
    g_mask = jnp.where(l_ix // 16 == r_ix % 8, 1.0 / 16.0, 0.0)
    degw_b = jnp.dot(row_exp * g_mask, jnp.ones((D, D), jnp.float32),
                     preferred_element_type=jnp.float32)
    o_ref[...] = acc + xw3c_ref[...] - degw_b * xw2_ref[...]


def _combine(acc2, wacc2, xw2, xw3c):
    grid = (NP // BN,)
    return pl.pallas_call(
        _combine_body,
        grid=grid,
        in_specs=[
            pl.BlockSpec((2, BN, D), lambda i: (0, i, 0)),
            pl.BlockSpec((2, QBN, D), lambda i: (0, i, 0)),
            pl.BlockSpec((BN, D), lambda i: (i, 0)),
            pl.BlockSpec((BN, D), lambda i: (i, 0)),
        ],
        out_specs=pl.BlockSpec((BN, D), lambda i: (i, 0)),
        out_shape=jax.ShapeDtypeStruct((NP, D), jnp.float32),
    )(acc2, wacc2, xw2, xw3c)


def kernel(x, edge_index, edge_weight, W1, b1, W2, W3, b3):
    src = edge_index[0].astype(jnp.int32).reshape(EROWS, 128)
    dst = edge_index[1].astype(jnp.int32).reshape(EROWS, 128)
    w = edge_weight.reshape(EROWS, 128)
    xp = jnp.pad(x, ((0, NP - N), (0, 0)))

    a, xw2, xw3c = _dense(xp, W1, b1, W2, W3, b3)

    sc_call = _make_sc_aggregate()(_sc_body)
    acc2, wacc2 = sc_call(a, src, dst, w)

    out = _combine(acc2, wacc2, xw2, xw3c)
    return out[:N]


# speedup vs baseline: 9.0804x; 1.4046x over previous
import functools

import jax
import jax.numpy as jnp
from jax import lax
from jax.experimental import pallas as pl
from jax.experimental.pallas import tpu as pltpu
from jax.experimental.pallas import tpu_sc as plsc

N = 10000
NP = 10240
NQR = NP // 128
D = 128
E = 320000
EROWS = E // 128
NC = 2
NS = 16
NW = NC * NS
RPT = NP // NS
NGROUPS = EROWS // 8
TAIL_ROWS = EROWS - NGROUPS * 8
BASE_GROUPS = NGROUPS // NW
EXTRA_GROUPS = NGROUPS - BASE_GROUPS * NW

BN = 1024
QRB = BN // 128


def _dense_body(x_ref, w1_ref, b1_ref, w2_ref, w3_ref, b3_ref,
                a_ref, xw2_ref, xw3c_ref):
    xb = x_ref[...]
    a_ref[...] = (jnp.dot(xb, w1_ref[...], preferred_element_type=jnp.float32)
                  + b1_ref[...])
    xw2_ref[...] = jnp.dot(xb, w2_ref[...], preferred_element_type=jnp.float32)
    xw3c_ref[...] = (jnp.dot(xb, w3_ref[...], preferred_element_type=jnp.float32)
                     + b3_ref[...])


def _dense(xp, W1, b1, W2, W3, b3):
    grid = (NP // BN,)
    return pl.pallas_call(
        _dense_body,
        grid=grid,
        in_specs=[
            pl.BlockSpec((BN, D), lambda i: (i, 0)),
            pl.BlockSpec((D, D), lambda i: (0, 0)),
            pl.BlockSpec((1, D), lambda i: (0, 0)),
            pl.BlockSpec((D, D), lambda i: (0, 0)),
            pl.BlockSpec((D, D), lambda i: (0, 0)),
            pl.BlockSpec((1, D), lambda i: (0, 0)),
        ],
        out_specs=[
            pl.BlockSpec((BN, D), lambda i: (i, 0)),
            pl.BlockSpec((BN, D), lambda i: (i, 0)),
            pl.BlockSpec((BN, D), lambda i: (i, 0)),
        ],
        out_shape=[
            jax.ShapeDtypeStruct((NP, D), jnp.float32),
            jax.ShapeDtypeStruct((NP, D), jnp.float32),
            jax.ShapeDtypeStruct((NP, D), jnp.float32),
        ],
    )(xp, W1, b1.reshape(1, D), W2, W3, b3.reshape(1, D))


def _sc_body(a_hbm, src_hbm, dst_hbm, w_hbm, acc_out, wacc_out,
             srcix_v, dstix_v, w_v, rows_a, rows_b, degw_v, acc_s,
             sem_a, sem_b):
    c = lax.axis_index("c")
    s = lax.axis_index("s")
    wid = c * NS + s

    def zbody(e, carry):
        for cg in range(8):
            rows_a[e, pl.ds(cg * 16, 16)] = jnp.zeros((16,), jnp.float32)
        return carry

    lax.fori_loop(0, 128, zbody, 0)

    def zdegw(e, carry):
        for cg in range(8):
            degw_v[e, pl.ds(cg * 16, 16)] = jnp.zeros((16,), jnp.float32)
        return carry

    lax.fori_loop(0, NQR, zdegw, 0)

    zbase = s * RPT
    for k in range(RPT // 128):
        pltpu.sync_copy(rows_a, acc_s.at[pl.ds(zbase + 128 * k, 128)])
    plsc.subcore_barrier()

    gstart = wid * BASE_GROUPS + jnp.minimum(wid, EXTRA_GROUPS)
    ngroups = BASE_GROUPS + jnp.where(wid < EXTRA_GROUPS, 1, 0)
    lane16 = lax.iota(jnp.int32, 16)

    def compute_chunk(k, buf):
        def ebody(g, ecarry):
            wv = w_v[k, pl.ds(g * 16, 16)]
            dv = dstix_v[k, pl.ds(g * 16, 16)]
            for r in range(16):
                ws = wv[r]
                dr = dv[r]
                e = g * 16 + r
                for cg in range(8):
                    sl = pl.ds(cg * 16, 16)
                    buf[e, sl] = buf[e, sl] * ws
                qrow = lax.shift_right_logical(dr, 7)
                qoff = (lax.shift_right_logical(dr, 4) & 7) * 16
                lane = dr & 15
                add = jnp.where(lane16 == lane, ws, 0.0)
                dsl = pl.ds(qoff, 16)
                degw_v[qrow, dsl] = degw_v[qrow, dsl] + add
            return ecarry

        lax.fori_loop(0, 8, ebody, 0)

    def load_group(grow, nk):
        pltpu.sync_copy(src_hbm.at[pl.ds(grow, nk)],
                        srcix_v.at[pl.ds(0, nk)])
        pltpu.sync_copy(dst_hbm.at[pl.ds(grow, nk)],
                        dstix_v.at[pl.ds(0, nk)])
        pltpu.sync_copy(w_hbm.at[pl.ds(grow, nk)], w_v.at[pl.ds(0, nk)])

    def run_group(nk):
        bufs = (rows_a, rows_b)
        sems = (sem_a, sem_b)
        pending = pltpu.async_copy(a_hbm.at[srcix_v.at[0]], bufs[0], sems[0])
        for k in range(nk):
            buf = bufs[k % 2]
            pending.wait()
            if k + 1 < nk:
                pending = pltpu.async_copy(a_hbm.at[srcix_v.at[k + 1]],
                                           bufs[(k + 1) % 2],
                                           sems[(k + 1) % 2])
            compute_chunk(k, buf)
            pltpu.sync_copy(buf, acc_s.at[dstix_v.at[k]], add=True)

    def group_body(j, carry):
        grow = pl.multiple_of((gstart + j) * 8, 8)
        load_group(grow, 8)
        run_group(8)
        return carry

    lax.fori_loop(0, ngroups, group_body, 0)

    if TAIL_ROWS:
        @pl.when(wid == NW - 1)
        def _tail():
            load_group(NGROUPS * 8, TAIL_ROWS)
            run_group(TAIL_ROWS)

    pltpu.sync_copy(degw_v, wacc_out.at[c, s])

    plsc.subcore_barrier()
    pltpu.sync_copy(acc_s.at[pl.ds(zbase, RPT)],
                    acc_out.at[c, pl.ds(zbase, RPT)])


def _make_sc_aggregate():
    mesh = plsc.VectorSubcoreMesh(core_axis_name="c", subcore_axis_name="s")
    return functools.partial(
        pl.kernel,
        mesh=mesh,
        out_type=[
            jax.ShapeDtypeStruct((NC, NP, D), jnp.float32),
            jax.ShapeDtypeStruct((NC, NS, NQR, D), jnp.float32),
        ],
        scratch_types=[
            pltpu.VMEM((8, 128), jnp.int32),
            pltpu.VMEM((8, 128), jnp.int32),
            pltpu.VMEM((8, 128), jnp.float32),
            pltpu.VMEM((128, D), jnp.float32),
            pltpu.VMEM((128, D), jnp.float32),
            pltpu.VMEM((NQR, D), jnp.float32),
            pltpu.VMEM_SHARED((NP, D), jnp.float32),
            pltpu.SemaphoreType.DMA,
            pltpu.SemaphoreType.DMA,
        ],
    )


def _combine_body(acc_ref, wacc_ref, xw2_ref, xw3c_ref, o_ref):
    acc = acc_ref[0] + acc_ref[1]
    w = wacc_ref[0, 0]
    for ci in range(NC):
        for si in range(NS):
            if ci or si:
                w = w + wacc_ref[ci, si]
    r16 = lax.broadcasted_iota(jnp.int32, (BN, QRB), 0)
    m16 = lax.broadcasted_iota(jnp.int32, (BN, QRB), 1)
    k1 = (m16 == lax.shift_right_logical(r16, 7)).astype(jnp.float32)
    row_exp = jnp.dot(k1, w, preferred_element_type=jnp.float32)
    r_ix = lax.broadcasted_iota(jnp.int32, (BN, D), 0)
    l_ix = lax.broadcasted_iota(jnp.int32, (BN, D), 1)
    msk = (l_ix == (r_ix & 127)).astype(jnp.float32)
    degw_b = jnp.dot(row_exp * msk, jnp.ones((D, D), jnp.float32),
                     preferred_element_type=jnp.float32)
    o_ref[...] = acc + xw3c_ref[...] - degw_b * xw2_ref[...]


def _combine(acc2, wacc2, xw2, xw3c):
    grid = (NP // BN,)
    return pl.pallas_call(
        _combine_body,
        grid=grid,
        in_specs=[
            pl.BlockSpec((2, BN, D), lambda i: (0, i, 0)),
            pl.BlockSpec((NC, NS, QRB, D), lambda i: (0, 0, i, 0)),
            pl.BlockSpec((BN, D), lambda i: (i, 0)),
            pl.BlockSpec((BN, D), lambda i: (i, 0)),
        ],
        out_specs=pl.BlockSpec((BN, D), lambda i: (i, 0)),
        out_shape=jax.ShapeDtypeStruct((NP, D), jnp.float32),
    )(acc2, wacc2, xw2, xw3c)


def kernel(x, edge_index, edge_weight, W1, b1, W2, W3, b3):
    src = edge_index[0].astype(jnp.int32).reshape(EROWS, 128)
    dst = edge_index[1].astype(jnp.int32).reshape(EROWS, 128)
    w = edge_weight.reshape(EROWS, 128)
    xp = jnp.pad(x, ((0, NP - N), (0, 0)))

    a, xw2, xw3c = _dense(xp, W1, b1, W2, W3, b3)

    sc_call = _make_sc_aggregate()(_sc_body)
    acc2, wacc2 = sc_call(a, src, dst, w)

    out = _combine(acc2, wacc2, xw2, xw3c)
    return out[:N]


# speedup vs baseline: 9.2807x; 1.0221x over previous
import functools

import jax
import jax.numpy as jnp
from jax import lax
from jax.experimental import pallas as pl
from jax.experimental.pallas import tpu as pltpu
from jax.experimental.pallas import tpu_sc as plsc

N = 10000
NP = 10240
NQR = NP // 128
D = 128
E = 320000
EROWS = E // 128
NC = 2
NS = 16
NW = NC * NS
RPT = NP // NS
NGROUPS = EROWS // 8
TAIL_ROWS = EROWS - NGROUPS * 8
BASE_GROUPS = NGROUPS // NW
EXTRA_GROUPS = NGROUPS - BASE_GROUPS * NW

BN = 1024
QRB = BN // 128


def _dense_body(x_ref, w1_ref, b1_ref, w2_ref, w3_ref, b3_ref,
                a_ref, xw2_ref, xw3c_ref):
    xb = x_ref[...]
    a_ref[...] = (jnp.dot(xb, w1_ref[...], preferred_element_type=jnp.float32)
                  + b1_ref[...])
    xw2_ref[...] = jnp.dot(xb, w2_ref[...], preferred_element_type=jnp.float32)
    xw3c_ref[...] = (jnp.dot(xb, w3_ref[...], preferred_element_type=jnp.float32)
                     + b3_ref[...])


def _dense(xp, W1, b1, W2, W3, b3):
    grid = (NP // BN,)
    return pl.pallas_call(
        _dense_body,
        grid=grid,
        in_specs=[
            pl.BlockSpec((BN, D), lambda i: (i, 0)),
            pl.BlockSpec((D, D), lambda i: (0, 0)),
            pl.BlockSpec((1, D), lambda i: (0, 0)),
            pl.BlockSpec((D, D), lambda i: (0, 0)),
            pl.BlockSpec((D, D), lambda i: (0, 0)),
            pl.BlockSpec((1, D), lambda i: (0, 0)),
        ],
        out_specs=[
            pl.BlockSpec((BN, D), lambda i: (i, 0)),
            pl.BlockSpec((BN, D), lambda i: (i, 0)),
            pl.BlockSpec((BN, D), lambda i: (i, 0)),
        ],
        out_shape=[
            jax.ShapeDtypeStruct((NP, D), jnp.float32),
            jax.ShapeDtypeStruct((NP, D), jnp.float32),
            jax.ShapeDtypeStruct((NP, D), jnp.float32),
        ],
    )(xp, W1, b1.reshape(1, D), W2, W3, b3.reshape(1, D))


def _sc_body(a_hbm, src_hbm, dst_hbm, w_hbm, acc_out, wacc_out,
             srcix_v, dstix_v, w_v, rows_a, rows_b, degw_v, acc_s,
             sem_a, sem_b, ssem_a, ssem_b):
    c = lax.axis_index("c")
    s = lax.axis_index("s")
    wid = c * NS + s

    def zbody(e, carry):
        for cg in range(8):
            rows_a[e, pl.ds(cg * 16, 16)] = jnp.zeros((16,), jnp.float32)
        return carry

    lax.fori_loop(0, 128, zbody, 0)

    def zdegw(e, carry):
        for cg in range(8):
            degw_v[e, pl.ds(cg * 16, 16)] = jnp.zeros((16,), jnp.float32)
        return carry

    lax.fori_loop(0, NQR, zdegw, 0)

    zbase = s * RPT
    for k in range(RPT // 128):
        pltpu.sync_copy(rows_a, acc_s.at[pl.ds(zbase + 128 * k, 128)])
    plsc.subcore_barrier()

    gstart = wid * BASE_GROUPS + jnp.minimum(wid, EXTRA_GROUPS)
    ngroups = BASE_GROUPS + jnp.where(wid < EXTRA_GROUPS, 1, 0)
    lane16 = lax.iota(jnp.int32, 16)

    def compute_chunk(k, buf):
        def ebody(g, ecarry):
            wv = w_v[k, pl.ds(g * 16, 16)]
            dv = dstix_v[k, pl.ds(g * 16, 16)]
            for r in range(16):
                ws = wv[r]
                dr = dv[r]
                e = g * 16 + r
                for cg in range(8):
                    sl = pl.ds(cg * 16, 16)
                    buf[e, sl] = buf[e, sl] * ws
                qrow = lax.shift_right_logical(dr, 7)
                qoff = (lax.shift_right_logical(dr, 4) & 7) * 16
                lane = dr & 15
                add = jnp.where(lane16 == lane, ws, 0.0)
                dsl = pl.ds(qoff, 16)
                degw_v[qrow, dsl] = degw_v[qrow, dsl] + add
            return ecarry

        lax.fori_loop(0, 8, ebody, 0)

    def load_group(grow, nk):
        pltpu.sync_copy(src_hbm.at[pl.ds(grow, nk)],
                        srcix_v.at[pl.ds(0, nk)])
        pltpu.sync_copy(dst_hbm.at[pl.ds(grow, nk)],
                        dstix_v.at[pl.ds(0, nk)])
        pltpu.sync_copy(w_hbm.at[pl.ds(grow, nk)], w_v.at[pl.ds(0, nk)])

    def run_group(nk):
        bufs = (rows_a, rows_b)
        sems = (sem_a, sem_b)
        ssems = (ssem_a, ssem_b)
        pend_g = pltpu.async_copy(a_hbm.at[srcix_v.at[0]], bufs[0], sems[0])
        pend_s = [None, None]
        for k in range(nk):
            buf = bufs[k % 2]
            pend_g.wait()
            if k + 1 < nk:
                if pend_s[(k + 1) % 2] is not None:
                    pend_s[(k + 1) % 2].wait()
                    pend_s[(k + 1) % 2] = None
                pend_g = pltpu.async_copy(a_hbm.at[srcix_v.at[k + 1]],
                                          bufs[(k + 1) % 2],
                                          sems[(k + 1) % 2])
            compute_chunk(k, buf)
            pend_s[k % 2] = pltpu.async_copy(buf, acc_s.at[dstix_v.at[k]],
                                             ssems[k % 2], add=True)
        for i in range(2):
            if pend_s[i] is not None:
                pend_s[i].wait()

    def group_body(j, carry):
        grow = pl.multiple_of((gstart + j) * 8, 8)
        load_group(grow, 8)
        run_group(8)
        return carry

    lax.fori_loop(0, ngroups, group_body, 0)

    if TAIL_ROWS:
        @pl.when(wid == NW - 1)
        def _tail():
            load_group(NGROUPS * 8, TAIL_ROWS)
            run_group(TAIL_ROWS)

    pltpu.sync_copy(degw_v, wacc_out.at[c, s])

    plsc.subcore_barrier()
    pltpu.sync_copy(acc_s.at[pl.ds(zbase, RPT)],
                    acc_out.at[c, pl.ds(zbase, RPT)])


def _make_sc_aggregate():
    mesh = plsc.VectorSubcoreMesh(core_axis_name="c", subcore_axis_name="s")
    return functools.partial(
        pl.kernel,
        mesh=mesh,
        out_type=[
            jax.ShapeDtypeStruct((NC, NP, D), jnp.float32),
            jax.ShapeDtypeStruct((NC, NS, NQR, D), jnp.float32),
        ],
        scratch_types=[
            pltpu.VMEM((8, 128), jnp.int32),
            pltpu.VMEM((8, 128), jnp.int32),
            pltpu.VMEM((8, 128), jnp.float32),
            pltpu.VMEM((128, D), jnp.float32),
            pltpu.VMEM((128, D), jnp.float32),
            pltpu.VMEM((NQR, D), jnp.float32),
            pltpu.VMEM_SHARED((NP, D), jnp.float32),
            pltpu.SemaphoreType.DMA,
            pltpu.SemaphoreType.DMA,
            pltpu.SemaphoreType.DMA,
            pltpu.SemaphoreType.DMA,
        ],
    )


def _combine_body(acc_ref, wacc_ref, xw2_ref, xw3c_ref, o_ref):
    acc = acc_ref[0] + acc_ref[1]
    w = wacc_ref[0, 0]
    for ci in range(NC):
        for si in range(NS):
            if ci or si:
                w = w + wacc_ref[ci, si]
    r16 = lax.broadcasted_iota(jnp.int32, (BN, QRB), 0)
    m16 = lax.broadcasted_iota(jnp.int32, (BN, QRB), 1)
    k1 = (m16 == lax.shift_right_logical(r16, 7)).astype(jnp.float32)
    row_exp = jnp.dot(k1, w, preferred_element_type=jnp.float32)
    r_ix = lax.broadcasted_iota(jnp.int32, (BN, D), 0)
    l_ix = lax.broadcasted_iota(jnp.int32, (BN, D), 1)
    msk = (l_ix == (r_ix & 127)).astype(jnp.float32)
    degw_b = jnp.dot(row_exp * msk, jnp.ones((D, D), jnp.float32),
                     preferred_element_type=jnp.float32)
    o_ref[...] = acc + xw3c_ref[...] - degw_b * xw2_ref[...]


def _combine(acc2, wacc2, xw2, xw3c):
    grid = (NP // BN,)
    return pl.pallas_call(
        _combine_body,
        grid=grid,
        in_specs=[
            pl.BlockSpec((2, BN, D), lambda i: (0, i, 0)),
            pl.BlockSpec((NC, NS, QRB, D), lambda i: (0, 0, i, 0)),
            pl.BlockSpec((BN, D), lambda i: (i, 0)),
            pl.BlockSpec((BN, D), lambda i: (i, 0)),
        ],
        out_specs=pl.BlockSpec((BN, D), lambda i: (i, 0)),
        out_shape=jax.ShapeDtypeStruct((NP, D), jnp.float32),
    )(acc2, wacc2, xw2, xw3c)


def kernel(x, edge_index, edge_weight, W1, b1, W2, W3, b3):
    src = edge_index[0].astype(jnp.int32).reshape(EROWS, 128)
    dst = edge_index[1].astype(jnp.int32).reshape(EROWS, 128)
    w = edge_weight.reshape(EROWS, 128)
    xp = jnp.pad(x, ((0, NP - N), (0, 0)))

    a, xw2, xw3c = _dense(xp, W1, b1, W2, W3, b3)

    sc_call = _make_sc_aggregate()(_sc_body)
    acc2, wacc2 = sc_call(a, src, dst, w)

    out = _combine(acc2, wacc2, xw2, xw3c)
    return out[:N]


# speedup vs baseline: 9.5437x; 1.0283x over previous
import functools

import jax
import jax.numpy as jnp
from jax import lax
from jax.experimental import pallas as pl
from jax.experimental.pallas import tpu as pltpu
from jax.experimental.pallas import tpu_sc as plsc

N = 10000
NP = 10240
NQR = NP // 128
D = 128
E = 320000
EROWS = E // 128
NC = 2
NS = 16
NW = NC * NS
RPT = NP // NS
NGROUPS = EROWS // 8
TAIL_ROWS = EROWS - NGROUPS * 8
BASE_GROUPS = NGROUPS // NW
EXTRA_GROUPS = NGROUPS - BASE_GROUPS * NW

CBN = 1000


def _dense_body(x_ref, w1_ref, b1_ref, a_ref):
    a_ref[...] = (jnp.dot(x_ref[...], w1_ref[...],
                          preferred_element_type=jnp.float32) + b1_ref[...])


def _dense(x, W1, b1):
    grid = (N // CBN,)
    return pl.pallas_call(
        _dense_body,
        grid=grid,
        in_specs=[
            pl.BlockSpec((CBN, D), lambda i: (i, 0)),
            pl.BlockSpec((D, D), lambda i: (0, 0)),
            pl.BlockSpec((1, D), lambda i: (0, 0)),
        ],
        out_specs=pl.BlockSpec((CBN, D), lambda i: (i, 0)),
        out_shape=jax.ShapeDtypeStruct((N, D), jnp.float32),
    )(x, W1, b1.reshape(1, D))


def _sc_body(a_hbm, src_hbm, dst_hbm, w_hbm, acc_out, wacc_out,
             srcix_v, dstix_v, w_v, rows_a, rows_b, degw_v, acc_s,
             sem_a, sem_b, ssem_a, ssem_b):
    c = lax.axis_index("c")
    s = lax.axis_index("s")
    wid = c * NS + s

    def zbody(e, carry):
        for cg in range(8):
            rows_a[e, pl.ds(cg * 16, 16)] = jnp.zeros((16,), jnp.float32)
        return carry

    lax.fori_loop(0, 128, zbody, 0)

    def zdegw(e, carry):
        for cg in range(8):
            degw_v[e, pl.ds(cg * 16, 16)] = jnp.zeros((16,), jnp.float32)
        return carry

    lax.fori_loop(0, NQR, zdegw, 0)

    zbase = s * RPT
    for k in range(RPT // 128):
        pltpu.sync_copy(rows_a, acc_s.at[pl.ds(zbase + 128 * k, 128)])
    plsc.subcore_barrier()

    gstart = wid * BASE_GROUPS + jnp.minimum(wid, EXTRA_GROUPS)
    ngroups = BASE_GROUPS + jnp.where(wid < EXTRA_GROUPS, 1, 0)
    lane16 = lax.iota(jnp.int32, 16)

    def compute_chunk(k, buf):
        def ebody(g, ecarry):
            wv = w_v[k, pl.ds(g * 16, 16)]
            dv = dstix_v[k, pl.ds(g * 16, 16)]
            for r in range(16):
                ws = wv[r]
                dr = dv[r]
                e = g * 16 + r
                for cg in range(8):
                    sl = pl.ds(cg * 16, 16)
                    buf[e, sl] = buf[e, sl] * ws
                qrow = lax.shift_right_logical(dr, 7)
                qoff = (lax.shift_right_logical(dr, 4) & 7) * 16
                lane = dr & 15
                add = jnp.where(lane16 == lane, ws, 0.0)
                dsl = pl.ds(qoff, 16)
                degw_v[qrow, dsl] = degw_v[qrow, dsl] + add
            return ecarry

        lax.fori_loop(0, 8, ebody, 0)

    def load_group(grow, nk):
        pltpu.sync_copy(src_hbm.at[pl.ds(grow, nk)],
                        srcix_v.at[pl.ds(0, nk)])
        pltpu.sync_copy(dst_hbm.at[pl.ds(grow, nk)],
                        dstix_v.at[pl.ds(0, nk)])
        pltpu.sync_copy(w_hbm.at[pl.ds(grow, nk)], w_v.at[pl.ds(0, nk)])

    def run_group(nk):
        bufs = (rows_a, rows_b)
        sems = (sem_a, sem_b)
        ssems = (ssem_a, ssem_b)
        pend_g = pltpu.async_copy(a_hbm.at[srcix_v.at[0]], bufs[0], sems[0])
        pend_s = [None, None]
        for k in range(nk):
            buf = bufs[k % 2]
            pend_g.wait()
            if k + 1 < nk:
                if pend_s[(k + 1) % 2] is not None:
                    pend_s[(k + 1) % 2].wait()
                    pend_s[(k + 1) % 2] = None
                pend_g = pltpu.async_copy(a_hbm.at[srcix_v.at[k + 1]],
                                          bufs[(k + 1) % 2],
                                          sems[(k + 1) % 2])
            compute_chunk(k, buf)
            pend_s[k % 2] = pltpu.async_copy(buf, acc_s.at[dstix_v.at[k]],
                                             ssems[k % 2], add=True)
        for i in range(2):
            if pend_s[i] is not None:
                pend_s[i].wait()

    def group_body(j, carry):
        grow = pl.multiple_of((gstart + j) * 8, 8)
        load_group(grow, 8)
        run_group(8)
        return carry

    lax.fori_loop(0, ngroups, group_body, 0)

    if TAIL_ROWS:
        @pl.when(wid == NW - 1)
        def _tail():
            load_group(NGROUPS * 8, TAIL_ROWS)
            run_group(TAIL_ROWS)

    pltpu.sync_copy(degw_v, wacc_out.at[c, s])

    plsc.subcore_barrier()
    pltpu.sync_copy(acc_s.at[pl.ds(zbase, RPT)],
                    acc_out.at[c, pl.ds(zbase, RPT)])


def _make_sc_aggregate():
    mesh = plsc.VectorSubcoreMesh(core_axis_name="c", subcore_axis_name="s")
    return functools.partial(
        pl.kernel,
        mesh=mesh,
        out_type=[
            jax.ShapeDtypeStruct((NC, NP, D), jnp.float32),
            jax.ShapeDtypeStruct((NC, NS, NQR, D), jnp.float32),
        ],
        scratch_types=[
            pltpu.VMEM((8, 128), jnp.int32),
            pltpu.VMEM((8, 128), jnp.int32),
            pltpu.VMEM((8, 128), jnp.float32),
            pltpu.VMEM((128, D), jnp.float32),
            pltpu.VMEM((128, D), jnp.float32),
            pltpu.VMEM((NQR, D), jnp.float32),
            pltpu.VMEM_SHARED((NP, D), jnp.float32),
            pltpu.SemaphoreType.DMA,
            pltpu.SemaphoreType.DMA,
            pltpu.SemaphoreType.DMA,
            pltpu.SemaphoreType.DMA,
        ],
    )


def _combine_body(acc_ref, wacc_ref, x_ref, w2_ref, w3_ref, b3_ref, o_ref):
    i = pl.program_id(0)
    acc = acc_ref[0] + acc_ref[1]
    w = wacc_ref[0, 0]
    for ci in range(NC):
        for si in range(NS):
            if ci or si:
                w = w + wacc_ref[ci, si]
    xb = x_ref[...]
    xw2 = jnp.dot(xb, w2_ref[...], preferred_element_type=jnp.float32)
    xw3c = (jnp.dot(xb, w3_ref[...], preferred_element_type=jnp.float32)
            + b3_ref[...])
    r16 = lax.broadcasted_iota(jnp.int32, (CBN, NQR), 0) + i * CBN
    m16 = lax.broadcasted_iota(jnp.int32, (CBN, NQR), 1)
    k1 = (m16 == lax.shift_right_logical(r16, 7)).astype(jnp.float32)
    row_exp = jnp.dot(k1, w, preferred_element_type=jnp.float32)
    r_ix = lax.broadcasted_iota(jnp.int32, (CBN, D), 0) + i * CBN
    l_ix = lax.broadcasted_iota(jnp.int32, (CBN, D), 1)
    msk = (l_ix == (r_ix & 127)).astype(jnp.float32)
    degw_b = jnp.dot(row_exp * msk, jnp.ones((D, D), jnp.float32),
                     preferred_element_type=jnp.float32)
    o_ref[...] = acc + xw3c - degw_b * xw2


def _combine(acc2, wacc2, x, W2, W3, b3):
    grid = (N // CBN,)
    return pl.pallas_call(
        _combine_body,
        grid=grid,
        in_specs=[
            pl.BlockSpec((2, CBN, D), lambda i: (0, i, 0)),
            pl.BlockSpec((NC, NS, NQR, D), lambda i: (0, 0, 0, 0)),
            pl.BlockSpec((CBN, D), lambda i: (i, 0)),
            pl.BlockSpec((D, D), lambda i: (0, 0)),
            pl.BlockSpec((D, D), lambda i: (0, 0)),
            pl.BlockSpec((1, D), lambda i: (0, 0)),
        ],
        out_specs=pl.BlockSpec((CBN, D), lambda i: (i, 0)),
        out_shape=jax.ShapeDtypeStruct((N, D), jnp.float32),
    )(acc2, wacc2, x, W2, W3, b3.reshape(1, D))


def kernel(x, edge_index, edge_weight, W1, b1, W2, W3, b3):
    src = edge_index[0].astype(jnp.int32).reshape(EROWS, 128)
    dst = edge_index[1].astype(jnp.int32).reshape(EROWS, 128)
    w = edge_weight.reshape(EROWS, 128)
    a = _dense(x, W1, b1)

    sc_call = _make_sc_aggregate()(_sc_body)
    acc2, wacc2 = sc_call(a, src, dst, w)

    return _combine(acc2, wacc2, x, W2, W3, b3)
